# baseline (speedup 1.0000x reference)
import functools

import jax
import jax.numpy as jnp
from jax import lax
from jax.experimental import pallas as pl
from jax.experimental.pallas import tpu as pltpu
from jax.experimental.pallas import tpu_sc as plsc

_KB = 128
_CH = 2048
_LANES = 16


def _sc_mesh():
    return plsc.VectorSubcoreMesh(core_axis_name="c", subcore_axis_name="s",
                                  num_cores=2, num_subcores=16)


def _make_sc_deg(nnp, e_pad):
    ep_t = e_pad // 16
    rows_t = ep_t // _KB
    n_chunks = ep_t // _CH
    cr = _CH // _KB
    stripe = nnp // 16

    def body(row_h, col_h, w_h, deg_out_h,
             deg_sh, crow_v, ccol_v, cw_v, w2d_v, zb_v):
        s = lax.axis_index("s")
        cc = lax.axis_index("c")
        rbase = s * rows_t

        def _zb(i, _):
            zb_v[pl.ds(i * _LANES, _LANES)] = jnp.zeros((_LANES,), jnp.float32)
            return 0
        lax.fori_loop(0, stripe // _LANES, _zb, 0)
        pltpu.sync_copy(zb_v, deg_sh.at[pl.ds(s * stripe, stripe)])
        plsc.subcore_barrier()

        for ch in range(n_chunks):
            base = rbase + ch * cr
            pltpu.sync_copy(row_h.at[pl.ds(base, cr)], crow_v)
            pltpu.sync_copy(col_h.at[pl.ds(base, cr)], ccol_v)
            pltpu.sync_copy(w_h.at[pl.ds(base, cr)], cw_v)

            def _deg(jj, _):
                for k in range(_KB // _LANES):
                    o = k * _LANES
                    r = crow_v[jj, pl.ds(o, _LANES)]
                    cl = ccol_v[jj, pl.ds(o, _LANES)]
                    wv = cw_v[jj, pl.ds(o, _LANES)]
                    w2d_v[jj, pl.ds(o, _LANES)] = jnp.where(r == cl, 0.0, wv)
                pltpu.sync_copy(w2d_v.at[jj], deg_sh.at[crow_v.at[jj]],
                                add=True)
                return 0
            lax.fori_loop(0, cr, _deg, 0)
        plsc.subcore_barrier()

        @pl.when(cc == 0)
        def _():
            pltpu.sync_copy(deg_sh.at[pl.ds(s * stripe, stripe)],
                            deg_out_h.at[pl.ds(s * stripe, stripe)])

    return pl.kernel(
        body,
        out_type=jax.ShapeDtypeStruct((nnp,), jnp.float32),
        mesh=_sc_mesh(),
        compiler_params=pltpu.CompilerParams(needs_layout_passes=False,
                                             use_tc_tiling_on_sc=False),
        scratch_types=[
            pltpu.VMEM_SHARED((nnp,), jnp.float32),
            pltpu.VMEM((cr, _KB), jnp.int32),
            pltpu.VMEM((cr, _KB), jnp.int32),
            pltpu.VMEM((cr, _KB), jnp.float32),
            pltpu.VMEM((cr, _KB), jnp.float32),
            pltpu.VMEM((stripe,), jnp.float32),
        ],
    )


def _dis_body(deg_ref, sc_ref, dis_ref):
    d = deg_ref[...]
    y = sc_ref[...] * lax.rsqrt(d)
    dis_ref[...] = jnp.where(d > 0.0, y, 0.0)


def _dis_tc(deg2d, scale2d):
    rows = deg2d.shape[0]
    return pl.pallas_call(
        _dis_body,
        grid=(1,),
        in_specs=[
            pl.BlockSpec((rows, 128), lambda i: (0, 0)),
            pl.BlockSpec((1, 128), lambda i: (0, 0)),
        ],
        out_specs=pl.BlockSpec((rows, 128), lambda i: (0, 0)),
        out_shape=jax.ShapeDtypeStruct((rows, 128), jnp.float32),
    )(deg2d, scale2d)


def _make_sc_scatter(num_nodes, n_per, l_steps, f_q, e_pad, r_rows, nnp):
    ep_t = e_pad // 16
    rows_t = ep_t // _KB
    n_chunks = ep_t // _CH
    cr = _CH // _KB
    acc_t = num_nodes // 16
    zr = next(z for z in range(min(64, acc_t), 0, -1) if acc_t % z == 0)
    nz = acc_t // zr
    o_st = (n_per // 16) // 8 * 8
    o_last = n_per - 15 * o_st
    assert n_per % 8 == 0 and o_last >= 0 and num_nodes == 2 * n_per
    boff = (l_steps - 1) * n_per

    def body(xq0_h, xq1_h, xq2_h, xq3_h, row_h, col_h, w_h, dis_h,
             tx0_h, tx1_h, tx2_h, tx3_h,
             acc_sh, slab_sh, dis_sh,
             wn_v, crow_v, ccol_v, cw_v, drb_v, dcb_v,
             rowb_v, colb_v, cidx_v, stage_v, sstage_v, idxg_v, zbuf_v,
             psem0, psem1, gsem0, gsem1, ssem0, ssem1, zsem, lsem):
        s = lax.axis_index("s")
        cc = lax.axis_index("c")
        rbase = s * rows_t
        psem = (psem0, psem1)
        gsem = (gsem0, gsem1)
        ssem = (ssem0, ssem1)

        @pl.when(s == 0)
        def _():
            pltpu.sync_copy(dis_h, dis_sh)

        def _z(i, _):
            for k in range(f_q // _LANES):
                zbuf_v[i, pl.ds(k * _LANES, _LANES)] = jnp.zeros(
                    (_LANES,), jnp.float32)
            return 0
        lax.fori_loop(0, zr, _z, 0)
        plsc.subcore_barrier()

        for ch in range(n_chunks):
            base = rbase + ch * cr
            pltpu.sync_copy(row_h.at[pl.ds(base, cr)], crow_v)
            pltpu.sync_copy(col_h.at[pl.ds(base, cr)], ccol_v)
            pltpu.sync_copy(w_h.at[pl.ds(base, cr)], cw_v)

            def _prep(jj, _):
                pltpu.sync_copy(dis_sh.at[crow_v.at[jj]], drb_v)
                pltpu.sync_copy(dis_sh.at[ccol_v.at[jj]], dcb_v)
                for k in range(_KB // _LANES):
                    o = k * _LANES
                    r = crow_v[jj, pl.ds(o, _LANES)]
                    cl = ccol_v[jj, pl.ds(o, _LANES)]
                    wv = cw_v[jj, pl.ds(o, _LANES)]
                    wn = -(drb_v[pl.ds(o, _LANES)] * wv
                           * dcb_v[pl.ds(o, _LANES)])
                    wn_v[ch * cr + jj, pl.ds(o, _LANES)] = jnp.where(
                        r == cl, 0.0, wn)
                return 0
            lax.fori_loop(0, cr, _prep, 0)

        def _zero_acc():
            for z in range(nz):
                pltpu.async_copy(
                    zbuf_v, acc_sh.at[pl.ds(s * acc_t + z * zr, zr)], zsem)

        def _zero_wait():
            for z in range(nz):
                pltpu.make_async_copy(
                    zbuf_v, acc_sh.at[pl.ds(s * acc_t + z * zr, zr)],
                    zsem).wait()

        def _fetch(p, j):
            pltpu.async_copy(row_h.at[rbase + j], rowb_v.at[p], psem[p])
            pltpu.async_copy(col_h.at[rbase + j], colb_v.at[p], psem[p])

        def _fetch_wait(p, j):
            pltpu.make_async_copy(row_h.at[rbase + j], rowb_v.at[p],
                                  psem[p]).wait()
            pltpu.make_async_copy(col_h.at[rbase + j], colb_v.at[p],
                                  psem[p]).wait()

        def _scat_wait(p):
            pltpu.make_async_copy(sstage_v.at[p], acc_sh.at[cidx_v.at[p]],
                                  ssem[p]).wait()

        def _launch(p, b):
            _fetch_wait(p, b)
            for k in range(_KB // _LANES):
                o = k * _LANES
                r = rowb_v[p, pl.ds(o, _LANES)]
                idxg_v[p, pl.ds(o, _LANES)] = r
                cidx_v[p, pl.ds(o, _LANES)] = colb_v[p, pl.ds(o, _LANES)]

            @pl.when(b + 2 < rows_t)
            def _():
                _fetch(p, b + 2)
            pltpu.async_copy(slab_sh.at[idxg_v.at[p]], stage_v.at[p],
                             gsem[p])

        def _process(p, b):
            pltpu.make_async_copy(slab_sh.at[idxg_v.at[p]],
                                  stage_v.at[p], gsem[p]).wait()
            bfull = jnp.full((_LANES,), b, jnp.int32)

            def _scale(e4, _):
                for u in range(4):
                    e = e4 * 4 + u
                    wns = plsc.load_gather(
                        wn_v, [bfull, jnp.full((_LANES,), e, jnp.int32)])
                    for k in range(f_q // (2 * _LANES)):
                        hv = stage_v[p, e, pl.ds(k * 2 * _LANES, 2 * _LANES)]
                        a, bb = plsc.unpack(
                            hv, format=plsc.PackFormat.INTERLEAVED)
                        o = k * 2 * _LANES
                        sstage_v[p, e, pl.ds(o, _LANES)] = a * wns
                        sstage_v[p, e, pl.ds(o + _LANES, _LANES)] = bb * wns
                return 0
            lax.fori_loop(0, _KB // 4, _scale, 0)
            pltpu.async_copy(sstage_v.at[p], acc_sh.at[cidx_v.at[p]],
                             ssem[p], add=True)

        def _qpass(l, xr_h, tr_h):
            local = s * acc_t
            src0 = l * n_per + local + jnp.where(local >= n_per, boff, 0)
            pltpu.async_copy(xr_h.at[pl.ds(src0, acc_t)],
                             slab_sh.at[pl.ds(local, acc_t)], lsem)
            _zero_acc()
            pltpu.make_async_copy(xr_h.at[pl.ds(src0, acc_t)],
                                  slab_sh.at[pl.ds(local, acc_t)],
                                  lsem).wait()
            _zero_wait()
            plsc.subcore_barrier()

            for p in range(2):
                _fetch(p, p)
            _launch(0, 0)

            def _iter(jj, _):
                @pl.when(jj > 0)
                def _():
                    _scat_wait(1)
                _launch(1, 2 * jj + 1)
                _process(0, 2 * jj)

                @pl.when(2 * jj + 2 < rows_t)
                def _():
                    _scat_wait(0)
                    _launch(0, 2 * jj + 2)
                _process(1, 2 * jj + 1)
                return 0
            lax.fori_loop(0, rows_t // 2, _iter, 0)
            for p in range(2):
                _scat_wait(p)
            plsc.subcore_barrier()

            src0o = s * o_st
            src1o = n_per + s * o_st
            dst0 = l * n_per + s * o_st
            dst1 = l_steps * n_per + l * n_per + s * o_st

            def _emit(rows):
                pltpu.sync_copy(acc_sh.at[pl.ds(src0o, rows)],
                                tr_h.at[pl.ds(dst0, rows)])
                pltpu.sync_copy(acc_sh.at[pl.ds(src1o, rows)],
                                tr_h.at[pl.ds(dst1, rows)])

            @pl.when(s < 15)
            def _():
                _emit(o_st)

            @pl.when(s == 15)
            def _():
                _emit(o_last)
            plsc.subcore_barrier()

        def _run(xa_h, xb_h, ta_h, tb_h):
            def _step(l, _):
                _qpass(l, xa_h, ta_h)
                _qpass(l, xb_h, tb_h)
                return 0
            lax.fori_loop(0, l_steps, _step, 0)

        @pl.when(cc == 0)
        def _():
            _run(xq0_h, xq1_h, tx0_h, tx1_h)

        @pl.when(cc == 1)
        def _():
            _run(xq2_h, xq3_h, tx2_h, tx3_h)

    out_sd = jax.ShapeDtypeStruct((r_rows, f_q), jnp.float32)
    return pl.kernel(
        body,
        out_type=(out_sd, out_sd, out_sd, out_sd),
        mesh=_sc_mesh(),
        compiler_params=pltpu.CompilerParams(needs_layout_passes=False,
                                             use_tc_tiling_on_sc=False),
        scratch_types=[
            pltpu.VMEM_SHARED((num_nodes, f_q), jnp.float32),
            pltpu.VMEM_SHARED((num_nodes, f_q), jnp.bfloat16),
            pltpu.VMEM_SHARED((nnp,), jnp.float32),
            pltpu.VMEM((rows_t, _KB), jnp.float32),
            pltpu.VMEM((cr, _KB), jnp.int32),
            pltpu.VMEM((cr, _KB), jnp.int32),
            pltpu.VMEM((cr, _KB), jnp.float32),
            pltpu.VMEM((_KB,), jnp.float32),
            pltpu.VMEM((_KB,), jnp.float32),
            pltpu.VMEM((2, _KB), jnp.int32),
            pltpu.VMEM((2, _KB), jnp.int32),
            pltpu.VMEM((2, _KB), jnp.int32),
            pltpu.VMEM((2, _KB, f_q), jnp.bfloat16),
            pltpu.VMEM((2, _KB, f_q), jnp.float32),
            pltpu.VMEM((2, _KB), jnp.int32),
            pltpu.VMEM((zr, f_q), jnp.float32),
            pltpu.SemaphoreType.DMA,
            pltpu.SemaphoreType.DMA,
            pltpu.SemaphoreType.DMA,
            pltpu.SemaphoreType.DMA,
            pltpu.SemaphoreType.DMA,
            pltpu.SemaphoreType.DMA,
            pltpu.SemaphoreType.DMA,
            pltpu.SemaphoreType.DMA,
        ],
    )


def _proj_body(hf_ref, t0_ref, t1_ref, t2_ref, t3_ref, wc_ref, b_ref,
               out_ref):
    f_in = hf_ref.shape[1]
    fq = t0_ref.shape[1]
    acc = jnp.dot(hf_ref[...], wc_ref[:f_in, :],
                  preferred_element_type=jnp.float32)
    for i, t_ref in enumerate((t0_ref, t1_ref, t2_ref, t3_ref)):
        acc += jnp.dot(t_ref[...],
                       wc_ref[f_in + i * fq:f_in + (i + 1) * fq, :],
                       preferred_element_type=jnp.float32)
    out_ref[...] = acc + b_ref[...]


def _fused_projection(hf, tq, wc, b):
    r, f_in = hf.shape
    fq = tq[0].shape[1]
    f_out = wc.shape[1]
    br = next(z for z in range(min(2400, r), 0, -8) if r % z == 0)
    return pl.pallas_call(
        _proj_body,
        grid=(r // br,),
        in_specs=[
            pl.BlockSpec((br, f_in), lambda i: (i, 0)),
            pl.BlockSpec((br, fq), lambda i: (i, 0)),
            pl.BlockSpec((br, fq), lambda i: (i, 0)),
            pl.BlockSpec((br, fq), lambda i: (i, 0)),
            pl.BlockSpec((br, fq), lambda i: (i, 0)),
            pl.BlockSpec((f_in + 4 * fq, f_out), lambda i: (0, 0)),
            pl.BlockSpec((f_out,), lambda i: (0,)),
        ],
        out_specs=pl.BlockSpec((br, f_out), lambda i: (i, 0)),
        out_shape=jax.ShapeDtypeStruct((r, f_out), jnp.float32),
    )(hf, *tq, wc, b)


def kernel(x, edge_index, edge_weight, W, b, lambda_max):
    lam = jnp.float32(2.0) if lambda_max is None else lambda_max
    batches, l_steps, n_per, f_in = x.shape
    num_nodes = batches * n_per
    f_q = f_in // 4
    f_out = W.shape[2]
    r_rows = batches * l_steps * n_per
    e = edge_weight.shape[0]
    nnp = ((num_nodes + 2047) // 2048) * 2048

    c = 2.0 / lam - 1.0
    f_q_w = W.shape[1] // 4
    w1_perm = jnp.concatenate(
        [jnp.concatenate([W[1][i * f_q_w:(i + 1) * f_q_w][0::2],
                          W[1][i * f_q_w:(i + 1) * f_q_w][1::2]])
         for i in range(4)])
    wc = jnp.concatenate([W[0] + c * W[1], w1_perm], axis=0)
    scale2d = jnp.full((1, 128), jnp.sqrt(2.0 / lam), jnp.float32)

    e_pad = ((e + 16 * _CH - 1) // (16 * _CH)) * (16 * _CH)
    pad = e_pad - e
    row = edge_index[0].astype(jnp.int32)
    col = edge_index[1].astype(jnp.int32)
    w = edge_weight.astype(jnp.float32)
    if pad:
        zi = jnp.zeros((pad,), jnp.int32)
        row = jnp.concatenate([row, zi])
        col = jnp.concatenate([col, zi])
        w = jnp.concatenate([w, jnp.zeros((pad,), jnp.float32)])
    row2 = row.reshape(e_pad // _KB, _KB)
    col2 = col.reshape(e_pad // _KB, _KB)
    w2 = w.reshape(e_pad // _KB, _KB)

    deg = _make_sc_deg(nnp, e_pad)(row2, col2, w2)
    dis = _dis_tc(deg.reshape(nnp // 128, 128), scale2d).reshape(nnp)

    xf = x.reshape(r_rows, f_in)
    xq = [xf[:, i * f_q:(i + 1) * f_q].astype(jnp.bfloat16)
          for i in range(4)]
    sc = _make_sc_scatter(num_nodes, n_per, l_steps, f_q, e_pad, r_rows, nnp)
    tq = sc(xq[0], xq[1], xq[2], xq[3], row2, col2, w2, dis)

    out_flat = _fused_projection(xf, list(tq), wc, b)
    return out_flat.reshape(batches, l_steps, n_per, f_out)


# speedup vs baseline: 1.0310x; 1.0310x over previous
import functools

import jax
import jax.numpy as jnp
from jax import lax
from jax.experimental import pallas as pl
from jax.experimental.pallas import tpu as pltpu
from jax.experimental.pallas import tpu_sc as plsc

_KB = 128
_CH = 2048
_LANES = 16


def _sc_mesh():
    return plsc.VectorSubcoreMesh(core_axis_name="c", subcore_axis_name="s",
                                  num_cores=2, num_subcores=16)


def _make_sc_deg(nnp, e_pad):
    ep_t = e_pad // 16
    rows_t = ep_t // _KB
    n_chunks = ep_t // _CH
    cr = _CH // _KB
    stripe = nnp // 16

    def body(row_h, col_h, w_h, deg_out_h,
             deg_sh, crow_v, ccol_v, cw_v, w2d_v, zb_v):
        s = lax.axis_index("s")
        cc = lax.axis_index("c")
        rbase = s * rows_t

        def _zb(i, _):
            zb_v[pl.ds(i * _LANES, _LANES)] = jnp.zeros((_LANES,), jnp.float32)
            return 0
        lax.fori_loop(0, stripe // _LANES, _zb, 0)
        pltpu.sync_copy(zb_v, deg_sh.at[pl.ds(s * stripe, stripe)])
        plsc.subcore_barrier()

        for ch in range(n_chunks):
            base = rbase + ch * cr
            pltpu.sync_copy(row_h.at[pl.ds(base, cr)], crow_v)
            pltpu.sync_copy(col_h.at[pl.ds(base, cr)], ccol_v)
            pltpu.sync_copy(w_h.at[pl.ds(base, cr)], cw_v)

            def _deg(jj, _):
                for k in range(_KB // _LANES):
                    o = k * _LANES
                    r = crow_v[jj, pl.ds(o, _LANES)]
                    cl = ccol_v[jj, pl.ds(o, _LANES)]
                    wv = cw_v[jj, pl.ds(o, _LANES)]
                    w2d_v[jj, pl.ds(o, _LANES)] = jnp.where(r == cl, 0.0, wv)
                pltpu.sync_copy(w2d_v.at[jj], deg_sh.at[crow_v.at[jj]],
                                add=True)
                return 0
            lax.fori_loop(0, cr, _deg, 0)
        plsc.subcore_barrier()

        @pl.when(cc == 0)
        def _():
            pltpu.sync_copy(deg_sh.at[pl.ds(s * stripe, stripe)],
                            deg_out_h.at[pl.ds(s * stripe, stripe)])

    return pl.kernel(
        body,
        out_type=jax.ShapeDtypeStruct((nnp,), jnp.float32),
        mesh=_sc_mesh(),
        compiler_params=pltpu.CompilerParams(needs_layout_passes=False,
                                             use_tc_tiling_on_sc=False),
        scratch_types=[
            pltpu.VMEM_SHARED((nnp,), jnp.float32),
            pltpu.VMEM((cr, _KB), jnp.int32),
            pltpu.VMEM((cr, _KB), jnp.int32),
            pltpu.VMEM((cr, _KB), jnp.float32),
            pltpu.VMEM((cr, _KB), jnp.float32),
            pltpu.VMEM((stripe,), jnp.float32),
        ],
    )


def _dis_body(deg_ref, sc_ref, dis_ref):
    d = deg_ref[...]
    y = sc_ref[...] * lax.rsqrt(d)
    dis_ref[...] = jnp.where(d > 0.0, y, 0.0)


def _dis_tc(deg2d, scale2d):
    rows = deg2d.shape[0]
    return pl.pallas_call(
        _dis_body,
        grid=(1,),
        in_specs=[
            pl.BlockSpec((rows, 128), lambda i: (0, 0)),
            pl.BlockSpec((1, 128), lambda i: (0, 0)),
        ],
        out_specs=pl.BlockSpec((rows, 128), lambda i: (0, 0)),
        out_shape=jax.ShapeDtypeStruct((rows, 128), jnp.float32),
    )(deg2d, scale2d)


def _make_sc_scatter(num_nodes, n_per, l_steps, f_q, e_pad, r_rows, nnp):
    ep_t = e_pad // 16
    rows_t = ep_t // _KB
    n_chunks = ep_t // _CH
    cr = _CH // _KB
    acc_t = num_nodes // 16
    zr = next(z for z in range(min(64, acc_t), 0, -1) if acc_t % z == 0)
    nz = acc_t // zr
    o_st = (n_per // 16) // 8 * 8
    o_last = n_per - 15 * o_st
    assert n_per % 8 == 0 and o_last >= 0 and num_nodes == 2 * n_per
    boff = (l_steps - 1) * n_per

    def body(xq0_h, xq1_h, xq2_h, xq3_h, row_h, col_h, w_h, dis_h,
             tx0_h, tx1_h, tx2_h, tx3_h,
             acc_sh, slab_sh, dis_sh,
             wn_v, crow_v, ccol_v, cw_v, drb_v, dcb_v,
             rowb_v, colb_v, cidx_v, stage_v, idxg_v, zbuf_v,
             psem0, psem1, gsem0, gsem1, ssem0, ssem1, zsem, lsem):
        s = lax.axis_index("s")
        cc = lax.axis_index("c")
        rbase = s * rows_t
        psem = (psem0, psem1)
        gsem = (gsem0, gsem1)
        ssem = (ssem0, ssem1)

        @pl.when(s == 0)
        def _():
            pltpu.sync_copy(dis_h, dis_sh)

        def _z(i, _):
            for k in range(f_q // _LANES):
                zbuf_v[i, pl.ds(k * _LANES, _LANES)] = jnp.zeros(
                    (_LANES,), jnp.float32)
            return 0
        lax.fori_loop(0, zr, _z, 0)
        plsc.subcore_barrier()

        for ch in range(n_chunks):
            base = rbase + ch * cr
            pltpu.sync_copy(row_h.at[pl.ds(base, cr)], crow_v)
            pltpu.sync_copy(col_h.at[pl.ds(base, cr)], ccol_v)
            pltpu.sync_copy(w_h.at[pl.ds(base, cr)], cw_v)

            def _prep(jj, _):
                pltpu.async_copy(dis_sh.at[crow_v.at[jj]], drb_v, gsem0)
                pltpu.async_copy(dis_sh.at[ccol_v.at[jj]], dcb_v, gsem1)
                pltpu.make_async_copy(dis_sh.at[crow_v.at[jj]], drb_v,
                                      gsem0).wait()
                pltpu.make_async_copy(dis_sh.at[ccol_v.at[jj]], dcb_v,
                                      gsem1).wait()
                for k in range(_KB // _LANES):
                    o = k * _LANES
                    r = crow_v[jj, pl.ds(o, _LANES)]
                    cl = ccol_v[jj, pl.ds(o, _LANES)]
                    wv = cw_v[jj, pl.ds(o, _LANES)]
                    wn = -(drb_v[pl.ds(o, _LANES)] * wv
                           * dcb_v[pl.ds(o, _LANES)])
                    wn_v[ch * cr + jj, pl.ds(o, _LANES)] = jnp.where(
                        r == cl, 0.0, wn)
                return 0
            lax.fori_loop(0, cr, _prep, 0)

        def _zero_acc():
            for z in range(nz):
                pltpu.async_copy(
                    zbuf_v, acc_sh.at[pl.ds(s * acc_t + z * zr, zr)], zsem)

        def _zero_wait():
            for z in range(nz):
                pltpu.make_async_copy(
                    zbuf_v, acc_sh.at[pl.ds(s * acc_t + z * zr, zr)],
                    zsem).wait()

        def _fetch(p, j):
            pltpu.async_copy(row_h.at[rbase + j], rowb_v.at[p], psem[p])
            pltpu.async_copy(col_h.at[rbase + j], colb_v.at[p], psem[p])

        def _fetch_wait(p, j):
            pltpu.make_async_copy(row_h.at[rbase + j], rowb_v.at[p],
                                  psem[p]).wait()
            pltpu.make_async_copy(col_h.at[rbase + j], colb_v.at[p],
                                  psem[p]).wait()

        def _scat_wait(p):
            pltpu.make_async_copy(stage_v.at[p], acc_sh.at[cidx_v.at[p]],
                                  ssem[p]).wait()

        def _launch(p, b):
            _fetch_wait(p, b)
            for k in range(_KB // _LANES):
                o = k * _LANES
                r = rowb_v[p, pl.ds(o, _LANES)]
                idxg_v[p, pl.ds(o, _LANES)] = r
                cidx_v[p, pl.ds(o, _LANES)] = colb_v[p, pl.ds(o, _LANES)]

            @pl.when(b + 2 < rows_t)
            def _():
                _fetch(p, b + 2)
            pltpu.async_copy(slab_sh.at[idxg_v.at[p]], stage_v.at[p],
                             gsem[p])

        def _process(p, b):
            pltpu.make_async_copy(slab_sh.at[idxg_v.at[p]],
                                  stage_v.at[p], gsem[p]).wait()
            bfull = jnp.full((_LANES,), b, jnp.int32)

            def _scale(e4, _):
                for u in range(4):
                    e = e4 * 4 + u
                    wns = plsc.load_gather(
                        wn_v, [bfull, jnp.full((_LANES,), e, jnp.int32)])
                    for k in range(f_q // _LANES):
                        sl = stage_v[p, e, pl.ds(k * _LANES, _LANES)]
                        stage_v[p, e, pl.ds(k * _LANES, _LANES)] = sl * wns
                return 0
            lax.fori_loop(0, _KB // 4, _scale, 0)
            pltpu.async_copy(stage_v.at[p], acc_sh.at[cidx_v.at[p]],
                             ssem[p], add=True)

        def _qpass(l, xr_h, tr_h):
            local = s * acc_t
            src0 = l * n_per + local + jnp.where(local >= n_per, boff, 0)
            pltpu.async_copy(xr_h.at[pl.ds(src0, acc_t)],
                             slab_sh.at[pl.ds(local, acc_t)], lsem)
            _zero_acc()
            pltpu.make_async_copy(xr_h.at[pl.ds(src0, acc_t)],
                                  slab_sh.at[pl.ds(local, acc_t)],
                                  lsem).wait()
            _zero_wait()
            plsc.subcore_barrier()

            for p in range(2):
                _fetch(p, p)
            _launch(0, 0)

            def _iter(jj, _):
                @pl.when(jj > 0)
                def _():
                    _scat_wait(1)
                _launch(1, 2 * jj + 1)
                _process(0, 2 * jj)

                @pl.when(2 * jj + 2 < rows_t)
                def _():
                    _scat_wait(0)
                    _launch(0, 2 * jj + 2)
                _process(1, 2 * jj + 1)
                return 0
            lax.fori_loop(0, rows_t // 2, _iter, 0)
            for p in range(2):
                _scat_wait(p)
            plsc.subcore_barrier()

            src0o = s * o_st
            src1o = n_per + s * o_st
            dst0 = l * n_per + s * o_st
            dst1 = l_steps * n_per + l * n_per + s * o_st

            def _emit(rows):
                pltpu.sync_copy(acc_sh.at[pl.ds(src0o, rows)],
                                tr_h.at[pl.ds(dst0, rows)])
                pltpu.sync_copy(acc_sh.at[pl.ds(src1o, rows)],
                                tr_h.at[pl.ds(dst1, rows)])

            @pl.when(s < 15)
            def _():
                _emit(o_st)

            @pl.when(s == 15)
            def _():
                _emit(o_last)
            plsc.subcore_barrier()

        def _run(xa_h, xb_h, ta_h, tb_h):
            def _step(l, _):
                _qpass(l, xa_h, ta_h)
                _qpass(l, xb_h, tb_h)
                return 0
            lax.fori_loop(0, l_steps, _step, 0)

        @pl.when(cc == 0)
        def _():
            _run(xq0_h, xq1_h, tx0_h, tx1_h)

        @pl.when(cc == 1)
        def _():
            _run(xq2_h, xq3_h, tx2_h, tx3_h)

    out_sd = jax.ShapeDtypeStruct((r_rows, f_q), jnp.float32)
    return pl.kernel(
        body,
        out_type=(out_sd, out_sd, out_sd, out_sd),
        mesh=_sc_mesh(),
        compiler_params=pltpu.CompilerParams(needs_layout_passes=False,
                                             use_tc_tiling_on_sc=False),
        scratch_types=[
            pltpu.VMEM_SHARED((num_nodes, f_q), jnp.float32),
            pltpu.VMEM_SHARED((num_nodes, f_q), jnp.float32),
            pltpu.VMEM_SHARED((nnp,), jnp.float32),
            pltpu.VMEM((rows_t, _KB), jnp.float32),
            pltpu.VMEM((cr, _KB), jnp.int32),
            pltpu.VMEM((cr, _KB), jnp.int32),
            pltpu.VMEM((cr, _KB), jnp.float32),
            pltpu.VMEM((_KB,), jnp.float32),
            pltpu.VMEM((_KB,), jnp.float32),
            pltpu.VMEM((2, _KB), jnp.int32),
            pltpu.VMEM((2, _KB), jnp.int32),
            pltpu.VMEM((2, _KB), jnp.int32),
            pltpu.VMEM((2, _KB, f_q), jnp.float32),
            pltpu.VMEM((2, _KB), jnp.int32),
            pltpu.VMEM((zr, f_q), jnp.float32),
            pltpu.SemaphoreType.DMA,
            pltpu.SemaphoreType.DMA,
            pltpu.SemaphoreType.DMA,
            pltpu.SemaphoreType.DMA,
            pltpu.SemaphoreType.DMA,
            pltpu.SemaphoreType.DMA,
            pltpu.SemaphoreType.DMA,
            pltpu.SemaphoreType.DMA,
        ],
    )


def _proj_body(hf_ref, t0_ref, t1_ref, t2_ref, t3_ref, wc_ref, b_ref,
               out_ref):
    f_in = hf_ref.shape[1]
    fq = t0_ref.shape[1]
    acc = jnp.dot(hf_ref[...], wc_ref[:f_in, :],
                  preferred_element_type=jnp.float32)
    for i, t_ref in enumerate((t0_ref, t1_ref, t2_ref, t3_ref)):
        acc += jnp.dot(t_ref[...],
                       wc_ref[f_in + i * fq:f_in + (i + 1) * fq, :],
                       preferred_element_type=jnp.float32)
    out_ref[...] = acc + b_ref[...]


def _fused_projection(hf, tq, wc, b):
    r, f_in = hf.shape
    fq = tq[0].shape[1]
    f_out = wc.shape[1]
    br = next(z for z in range(min(2400, r), 0, -8) if r % z == 0)
    return pl.pallas_call(
        _proj_body,
        grid=(r // br,),
        in_specs=[
            pl.BlockSpec((br, f_in), lambda i: (i, 0)),
            pl.BlockSpec((br, fq), lambda i: (i, 0)),
            pl.BlockSpec((br, fq), lambda i: (i, 0)),
            pl.BlockSpec((br, fq), lambda i: (i, 0)),
            pl.BlockSpec((br, fq), lambda i: (i, 0)),
            pl.BlockSpec((f_in + 4 * fq, f_out), lambda i: (0, 0)),
            pl.BlockSpec((f_out,), lambda i: (0,)),
        ],
        out_specs=pl.BlockSpec((br, f_out), lambda i: (i, 0)),
        out_shape=jax.ShapeDtypeStruct((r, f_out), jnp.float32),
    )(hf, *tq, wc, b)


def kernel(x, edge_index, edge_weight, W, b, lambda_max):
    lam = jnp.float32(2.0) if lambda_max is None else lambda_max
    batches, l_steps, n_per, f_in = x.shape
    num_nodes = batches * n_per
    f_q = f_in // 4
    f_out = W.shape[2]
    r_rows = batches * l_steps * n_per
    e = edge_weight.shape[0]
    nnp = ((num_nodes + 2047) // 2048) * 2048

    c = 2.0 / lam - 1.0
    wc = jnp.concatenate([W[0] + c * W[1], W[1]], axis=0)
    scale2d = jnp.full((1, 128), jnp.sqrt(2.0 / lam), jnp.float32)

    e_pad = ((e + 16 * _CH - 1) // (16 * _CH)) * (16 * _CH)
    pad = e_pad - e
    row = edge_index[0].astype(jnp.int32)
    col = edge_index[1].astype(jnp.int32)
    w = edge_weight.astype(jnp.float32)
    if pad:
        zi = jnp.zeros((pad,), jnp.int32)
        row = jnp.concatenate([row, zi])
        col = jnp.concatenate([col, zi])
        w = jnp.concatenate([w, jnp.zeros((pad,), jnp.float32)])
    row2 = row.reshape(e_pad // _KB, _KB)
    col2 = col.reshape(e_pad // _KB, _KB)
    w2 = w.reshape(e_pad // _KB, _KB)

    deg = _make_sc_deg(nnp, e_pad)(row2, col2, w2)
    dis = _dis_tc(deg.reshape(nnp // 128, 128), scale2d).reshape(nnp)

    xf = x.reshape(r_rows, f_in)
    xq = [xf[:, i * f_q:(i + 1) * f_q] for i in range(4)]
    sc = _make_sc_scatter(num_nodes, n_per, l_steps, f_q, e_pad, r_rows, nnp)
    tq = sc(xq[0], xq[1], xq[2], xq[3], row2, col2, w2, dis)

    out_flat = _fused_projection(xf, list(tq), wc, b)
    return out_flat.reshape(batches, l_steps, n_per, f_out)
